# baseline (speedup 1.0000x reference)
import functools

import jax
import jax.numpy as jnp
from jax import lax
from jax.experimental import pallas as pl
from jax.experimental.pallas import tpu as pltpu
from jax.experimental.pallas import tpu_sc as plsc

B = 16384
D = 64
L = 16
CW = 128
NCH = 7813
NE = 2 * B
RING = 16
TAILOFF = (NCH - 1) * CW


def _shr(x, n):
    return jax.lax.shift_right_logical(x, n)


def _make_scan_kernel(num_cores, num_subcores):
    NW = num_cores * num_subcores
    mesh = plsc.VectorSubcoreMesh(core_axis_name="c", subcore_axis_name="s")

    @functools.partial(
        pl.kernel,
        mesh=mesh,
        out_type=jax.ShapeDtypeStruct(((NE + 1) * D,), jnp.float32),
        scratch_types=[
            pltpu.VMEM((2048,), jnp.int32),
            pltpu.VMEM((NE + L,), jnp.int32),
            pltpu.VMEM((2 * D * CW,), jnp.float32),
            pltpu.VMEM((8, 2048), jnp.float32),
            pltpu.VMEM((8, 2048), jnp.float32),
            pltpu.VMEM((RING * D,), jnp.float32),
            pltpu.VMEM((2 * D, D), jnp.float32),
            pltpu.VMEM((256,), jnp.int32),
            pltpu.VMEM((256,), jnp.int32),
            pltpu.VMEM((256,), jnp.int32),
            pltpu.SMEM((256,), jnp.int32),
            pltpu.SMEM((256,), jnp.int32),
            pltpu.SemaphoreType.DMA,
            pltpu.SemaphoreType.DMA,
            pltpu.SemaphoreType.DMA,
        ],
        compiler_params=pltpu.CompilerParams(needs_layout_passes=False),
    )
    def k1(uidx_hbm, midx_hbm, utT_hbm, mtT_hbm, stage_hbm,
           piece_v, plist_v, cbuf_v, slabA_v, slabB_v, ring_v, tbuf_v,
           cnt_v, base_v, bw_v, sbase_s, scnt_s,
           semA, semB, sem_r):
        w = lax.axis_index("s") * num_cores + lax.axis_index("c")
        riota = lax.iota(jnp.int32, L)
        zeros16 = jnp.zeros((L,), jnp.int32)
        ones16 = jnp.full((L,), 1, jnp.int32)

        for k in range(16):
            cnt_v[pl.ds(k * L, L)] = zeros16

        def scan(place):
            for tab in range(2):
                idx_hbm = uidx_hbm if tab == 0 else midx_hbm
                for p in range(8):
                    pltpu.sync_copy(idx_hbm.at[pl.ds(p * 2048, 2048)],
                                    piece_v)

                    def svec(i, carry):
                        rv = piece_v[pl.ds(i * L, L)]
                        cid = _shr(rv, 7)
                        mine = (cid & (NW - 1)) == w
                        qv = _shr(cid, 5)
                        if not place:
                            plsc.addupdate_scatter(cnt_v, [qv], ones16,
                                                   mask=mine)
                        else:
                            bkey = (p * 2048 + i * L + riota) * 2 + tab
                            pk = bkey * CW + (rv & (CW - 1))
                            rank = plsc.scan_count(qv, mask=mine)[0]
                            posv = (plsc.load_gather(bw_v, [qv])
                                    + rank - 1)
                            plsc.store_scatter(plist_v, [posv], pk,
                                               mask=mine)
                            plsc.addupdate_scatter(bw_v, [qv], ones16,
                                                   mask=mine)
                        return carry

                    lax.fori_loop(0, 128, svec, 0)

        scan(False)

        carry = 0
        for k in range(16):
            ck = cnt_v[pl.ds(k * L, L)]
            incl = plsc.cumsum(ck)
            base_v[pl.ds(k * L, L)] = incl - ck + carry
            carry = carry + incl[L - 1]
        for k in range(16):
            bw_v[pl.ds(k * L, L)] = base_v[pl.ds(k * L, L)]

        scan(True)

        for k in range(16):
            bk = base_v[pl.ds(k * L, L)]
            ck = cnt_v[pl.ds(k * L, L)]
            for l in range(L):
                sbase_s[k * L + l] = bk[l]
                scnt_s[k * L + l] = ck[l]

        pq = [(q * L + riota) * CW for q in range(4)]

        def extract_bucket(q, mcnt):
            lo = sbase_s[q]
            n = scnt_s[q]
            nv = _shr(n + L - 1, 4)

            def vbody(v, mc):
                pkv = plist_v[pl.ds(lo + v * L, L)]
                valid = riota < (n - v * L)
                bsafe = jnp.where(valid, _shr(pkv, 7), NE)
                rcv = pkv & (CW - 1)
                for j in range(L):
                    bkey = bsafe[j]
                    rc = rcv[j]
                    tab = bkey & 1
                    base = tab * (D * CW) + rc
                    slot = mc & (RING - 1)

                    @pl.when(mc >= RING)
                    def _():
                        pltpu.make_async_copy(
                            ring_v.at[pl.ds(0, D)],
                            stage_hbm.at[pl.ds(0, D)], sem_r).wait()

                    for q4 in range(4):
                        gv = plsc.load_gather(cbuf_v, [pq[q4] + base])
                        ring_v[pl.ds(slot * D + q4 * L, L)] = gv
                    pltpu.make_async_copy(
                        ring_v.at[pl.ds(slot * D, D)],
                        stage_hbm.at[pl.ds(bkey * D, D)], sem_r).start()
                    mc = mc + 1
                return mc

            mcnt = lax.fori_loop(0, nv, vbody, mcnt)

            def dbody(i, carry2):
                pltpu.make_async_copy(
                    ring_v.at[pl.ds(0, D)],
                    stage_hbm.at[pl.ds(0, D)], sem_r).wait()
                return carry2

            lax.fori_loop(0, jnp.minimum(mcnt, RING), dbody, 0)
            return 0

        nreg = lax.select(w < NCH - NW * (NCH // NW), NCH // NW + 1,
                          NCH // NW)
        nreg = lax.select(w == (NCH - 1) % NW, nreg - 1, nreg)

        def issue(i, slab, sem):
            cid = w + NW * i
            off = pl.multiple_of(cid * CW, CW)
            for tab in range(2):
                src = utT_hbm if tab == 0 else mtT_hbm
                for db in range(8):
                    pltpu.make_async_copy(
                        src.at[pl.ds(8 * db, 8), pl.ds(off, CW)],
                        slab.at[pl.ds(0, 8),
                                pl.ds((tab * 8 + db) * CW, CW)],
                        sem).start()

        def wait_slab(slab, sem):
            for _ in range(16):
                pltpu.make_async_copy(
                    utT_hbm.at[pl.ds(0, 8), pl.ds(0, CW)],
                    slab.at[pl.ds(0, 8), pl.ds(0, CW)], sem).wait()

        def detile(slab):
            def dt(j, carry):
                for tab in range(2):
                    for db in range(8):
                        for c in range(8):
                            vec = slab[j, pl.ds((tab * 8 + db) * CW
                                                + c * L, L)]
                            cbuf_v[pl.ds(tab * (D * CW)
                                         + (db * 8 + j) * CW + c * L,
                                         L)] = vec
                return carry

            lax.fori_loop(0, 8, dt, 0)

        @pl.when(nreg > 0)
        def _():
            issue(0, slabA_v, semA)

        @pl.when(nreg > 1)
        def _():
            issue(1, slabB_v, semB)

        ng = _shr(nreg + 1, 1)

        def gbody(g, carry):
            for par in range(2):
                i = g * 2 + par
                slab = slabA_v if par == 0 else slabB_v
                sem = semA if par == 0 else semB

                @pl.when(i < nreg)
                def _():
                    wait_slab(slab, sem)
                    detile(slab)

                    @pl.when(i + 2 < nreg)
                    def _():
                        issue(i + 2, slab, sem)

                    extract_bucket(i, 0)
            return carry

        lax.fori_loop(0, ng, gbody, 0)

        @pl.when(w == (NCH - 1) % NW)
        def _():
            twid = 1000000 - TAILOFF

            def tissue(d8, carry):
                for dj in range(8):
                    d = d8 * 8 + dj
                    pltpu.make_async_copy(
                        utT_hbm.at[pl.ds(d, 1), pl.ds(TAILOFF, twid)],
                        tbuf_v.at[pl.ds(d, 1)], semA).start()
                    pltpu.make_async_copy(
                        mtT_hbm.at[pl.ds(d, 1), pl.ds(TAILOFF, twid)],
                        tbuf_v.at[pl.ds(D + d, 1)], semA).start()
                return carry

            lax.fori_loop(0, D // 8, tissue, 0)

            def tdrain(d8, carry):
                for dj in range(2):
                    pltpu.make_async_copy(
                        utT_hbm.at[pl.ds(0, 1), pl.ds(TAILOFF, twid)],
                        tbuf_v.at[pl.ds(0, 1)], semA).wait()
                return carry

            lax.fori_loop(0, D, tdrain, 0)

            for tab in range(2):
                for d in range(D):
                    for c in range(4):
                        vec = tbuf_v[tab * D + d, pl.ds(c * L, L)]
                        cbuf_v[pl.ds(tab * (D * CW) + d * CW + c * L,
                                     L)] = vec
            extract_bucket((NCH - 1) >> 5, 0)

    return k1


def _make_dot_kernel(num_cores, num_subcores):
    NW = num_cores * num_subcores
    bw = B // NW
    mesh = plsc.VectorSubcoreMesh(core_axis_name="c", subcore_axis_name="s")

    @functools.partial(
        pl.kernel,
        mesh=mesh,
        out_type=jax.ShapeDtypeStruct((B,), jnp.float32),
        scratch_types=[
            pltpu.VMEM((bw * 2 * D,), jnp.float32),
            pltpu.VMEM((bw,), jnp.float32),
            pltpu.VMEM((L * L,), jnp.float32),
        ],
        compiler_params=pltpu.CompilerParams(needs_layout_passes=False),
    )
    def k2(stage_hbm, out_hbm, flat_v, out_v, accbuf_v):
        wid = lax.axis_index("s") * num_cores + lax.axis_index("c")
        base = wid * bw
        pltpu.sync_copy(stage_hbm.at[pl.ds(base * 2 * D, bw * 2 * D)], flat_v)
        riota = lax.iota(jnp.int32, L)

        def body(g, carry):
            for j in range(L):
                p = (g * L + j) * 2 * D
                acc = flat_v[pl.ds(p, L)] * flat_v[pl.ds(p + D, L)]
                for q in range(1, D // L):
                    acc = acc + (flat_v[pl.ds(p + q * L, L)]
                                 * flat_v[pl.ds(p + D + q * L, L)])
                accbuf_v[pl.ds(j * L, L)] = acc
            res = jnp.zeros((L,), jnp.float32)
            for i in range(L):
                res = res + plsc.load_gather(accbuf_v, [riota * L + i])
            out_v[pl.ds(g * L, L)] = res
            return carry

        lax.fori_loop(0, bw // L, body, 0)
        pltpu.sync_copy(out_v, out_hbm.at[pl.ds(base, bw)])

    return k2


def kernel(inputs, user_table, movie_table):
    info = plsc.get_sparse_core_info()
    k1 = _make_scan_kernel(info.num_cores, info.num_subcores)
    k2 = _make_dot_kernel(info.num_cores, info.num_subcores)
    user_idx = inputs[:, 0]
    movie_idx = inputs[:, 1]
    stage = k1(user_idx, movie_idx, user_table.T, movie_table.T)
    out = k2(stage)
    return out.reshape(B, 1)


# speedup vs baseline: 2.0452x; 2.0452x over previous
import functools

import jax
import jax.numpy as jnp
from jax import lax
from jax.experimental import pallas as pl
from jax.experimental.pallas import tpu as pltpu
from jax.experimental.pallas import tpu_sc as plsc

B = 16384
D = 64
L = 16
CW = 256
CSH = 8
NCH = 3907
NE = 2 * B
RING = 16
TAILOFF = (NCH - 1) * CW


def _shr(x, n):
    return jax.lax.shift_right_logical(x, n)


def _make_scan_kernel(num_cores, num_subcores):
    NW = num_cores * num_subcores
    mesh = plsc.VectorSubcoreMesh(core_axis_name="c", subcore_axis_name="s")

    @functools.partial(
        pl.kernel,
        mesh=mesh,
        out_type=jax.ShapeDtypeStruct(((NE + 1) * D,), jnp.float32),
        scratch_types=[
            pltpu.VMEM((2048,), jnp.int32),
            pltpu.VMEM((NE + L,), jnp.int32),
            pltpu.VMEM((2 * D * CW,), jnp.float32),
            pltpu.VMEM((2 * D * CW,), jnp.float32),
            pltpu.VMEM((RING * D,), jnp.float32),
            pltpu.VMEM((2 * D, D), jnp.float32),
            pltpu.VMEM((256,), jnp.int32),
            pltpu.VMEM((256,), jnp.int32),
            pltpu.VMEM((256,), jnp.int32),
            pltpu.SMEM((256,), jnp.int32),
            pltpu.SMEM((256,), jnp.int32),
            pltpu.SemaphoreType.DMA,
            pltpu.SemaphoreType.DMA,
            pltpu.SemaphoreType.DMA,
        ],
        compiler_params=pltpu.CompilerParams(needs_layout_passes=False),
    )
    def k1(uidx_hbm, midx_hbm, utT_hbm, mtT_hbm, stage_hbm,
           piece_v, plist_v, cbufA_v, cbufB_v, ring_v, tbuf_v,
           cnt_v, base_v, bw_v, sbase_s, scnt_s,
           semA, semB, sem_r):
        w = lax.axis_index("s") * num_cores + lax.axis_index("c")
        riota = lax.iota(jnp.int32, L)
        zeros16 = jnp.zeros((L,), jnp.int32)
        ones16 = jnp.full((L,), 1, jnp.int32)

        for k in range(16):
            cnt_v[pl.ds(k * L, L)] = zeros16

        def scan(place):
            for tab in range(2):
                idx_hbm = uidx_hbm if tab == 0 else midx_hbm
                for p in range(8):
                    pltpu.sync_copy(idx_hbm.at[pl.ds(p * 2048, 2048)],
                                    piece_v)

                    def svec(i, carry):
                        rv = piece_v[pl.ds(i * L, L)]
                        cid = _shr(rv, CSH)
                        mine = (cid & (NW - 1)) == w
                        qv = _shr(cid, 5)
                        if not place:
                            plsc.addupdate_scatter(cnt_v, [qv], ones16,
                                                   mask=mine)
                        else:
                            bkey = (p * 2048 + i * L + riota) * 2 + tab
                            pk = bkey * CW + (rv & (CW - 1))
                            rank = plsc.scan_count(qv, mask=mine)[0]
                            posv = (plsc.load_gather(bw_v, [qv])
                                    + rank - 1)
                            plsc.store_scatter(plist_v, [posv], pk,
                                               mask=mine)
                            plsc.addupdate_scatter(bw_v, [qv], ones16,
                                                   mask=mine)
                        return carry

                    lax.fori_loop(0, 128, svec, 0)

        scan(False)

        carry = 0
        for k in range(16):
            ck = cnt_v[pl.ds(k * L, L)]
            incl = plsc.cumsum(ck)
            base_v[pl.ds(k * L, L)] = incl - ck + carry
            carry = carry + incl[L - 1]
        for k in range(16):
            bw_v[pl.ds(k * L, L)] = base_v[pl.ds(k * L, L)]

        scan(True)

        for k in range(16):
            bk = base_v[pl.ds(k * L, L)]
            ck = cnt_v[pl.ds(k * L, L)]
            for l in range(L):
                sbase_s[k * L + l] = bk[l]
                scnt_s[k * L + l] = ck[l]

        pq = [(q * L + riota) * CW for q in range(4)]

        def extract_bucket(q, mcnt, cbuf_v):
            lo = sbase_s[q]
            n = scnt_s[q]
            nv = _shr(n + L - 1, 4)

            def vbody(v, mc):
                pkv = plist_v[pl.ds(lo + v * L, L)]
                valid = riota < (n - v * L)
                bsafe = jnp.where(valid, _shr(pkv, CSH), NE)
                rcv = pkv & (CW - 1)
                for j in range(L):
                    bkey = bsafe[j]
                    rc = rcv[j]
                    tab = bkey & 1
                    base = tab * (D * CW) + rc
                    slot = mc & (RING - 1)

                    @pl.when(mc >= RING)
                    def _():
                        pltpu.make_async_copy(
                            ring_v.at[pl.ds(0, D)],
                            stage_hbm.at[pl.ds(0, D)], sem_r).wait()

                    for q4 in range(4):
                        gv = plsc.load_gather(cbuf_v, [pq[q4] + base])
                        ring_v[pl.ds(slot * D + q4 * L, L)] = gv
                    pltpu.make_async_copy(
                        ring_v.at[pl.ds(slot * D, D)],
                        stage_hbm.at[pl.ds(bkey * D, D)], sem_r).start()
                    mc = mc + 1
                return mc

            mcnt = lax.fori_loop(0, nv, vbody, mcnt)

            def dbody(i, carry2):
                pltpu.make_async_copy(
                    ring_v.at[pl.ds(0, D)],
                    stage_hbm.at[pl.ds(0, D)], sem_r).wait()
                return carry2

            lax.fori_loop(0, jnp.minimum(mcnt, RING), dbody, 0)
            return 0

        nreg = lax.select(w < NCH - NW * (NCH // NW), NCH // NW + 1,
                          NCH // NW)
        nreg = lax.select(w == (NCH - 1) % NW, nreg - 1, nreg)

        def issue(i, cbuf, sem):
            cid = w + NW * i
            off = pl.multiple_of(cid * CW, CW)

            def di(d8, carry):
                for dj in range(8):
                    d = d8 * 8 + dj
                    pltpu.make_async_copy(
                        utT_hbm.at[d, pl.ds(off, CW)],
                        cbuf.at[pl.ds(d * CW, CW)], sem).start()
                    pltpu.make_async_copy(
                        mtT_hbm.at[d, pl.ds(off, CW)],
                        cbuf.at[pl.ds(D * CW + d * CW, CW)], sem).start()
                return carry

            lax.fori_loop(0, D // 8, di, 0)

        def wait_chunk(cbuf, sem):
            def dw(d8, carry):
                for dj in range(2):
                    pltpu.make_async_copy(
                        utT_hbm.at[0, pl.ds(0, CW)],
                        cbuf.at[pl.ds(0, CW)], sem).wait()
                return carry

            lax.fori_loop(0, D, dw, 0)

        issue(0, cbufA_v, semA)
        issue(1, cbufB_v, semB)
        ng = _shr(nreg, 1)

        def gbody(g, carry):
            for par in range(2):
                i = g * 2 + par
                cbuf = cbufA_v if par == 0 else cbufB_v
                sem = semA if par == 0 else semB
                wait_chunk(cbuf, sem)
                extract_bucket(i, 0, cbuf)

                @pl.when(i + 2 < nreg)
                def _():
                    issue(i + 2, cbuf, sem)
            return carry

        lax.fori_loop(0, ng, gbody, 0)

        @pl.when((nreg & 1) != 0)
        def _():
            wait_chunk(cbufA_v, semA)
            extract_bucket(nreg - 1, 0, cbufA_v)

        @pl.when(w == (NCH - 1) % NW)
        def _():
            twid = 1000000 - TAILOFF

            def tissue(d8, carry):
                for dj in range(8):
                    d = d8 * 8 + dj
                    pltpu.make_async_copy(
                        utT_hbm.at[pl.ds(d, 1), pl.ds(TAILOFF, twid)],
                        tbuf_v.at[pl.ds(d, 1)], semA).start()
                    pltpu.make_async_copy(
                        mtT_hbm.at[pl.ds(d, 1), pl.ds(TAILOFF, twid)],
                        tbuf_v.at[pl.ds(D + d, 1)], semA).start()
                return carry

            lax.fori_loop(0, D // 8, tissue, 0)

            def tdrain(d8, carry):
                for dj in range(2):
                    pltpu.make_async_copy(
                        utT_hbm.at[pl.ds(0, 1), pl.ds(TAILOFF, twid)],
                        tbuf_v.at[pl.ds(0, 1)], semA).wait()
                return carry

            lax.fori_loop(0, D, tdrain, 0)

            for tab in range(2):
                for d in range(D):
                    for c in range(4):
                        vec = tbuf_v[tab * D + d, pl.ds(c * L, L)]
                        cbufA_v[pl.ds(tab * (D * CW) + d * CW + c * L,
                                      L)] = vec
            extract_bucket((NCH - 1) >> 5, 0, cbufA_v)

    return k1


def _make_dot_kernel(num_cores, num_subcores):
    NW = num_cores * num_subcores
    bw = B // NW
    mesh = plsc.VectorSubcoreMesh(core_axis_name="c", subcore_axis_name="s")

    @functools.partial(
        pl.kernel,
        mesh=mesh,
        out_type=jax.ShapeDtypeStruct((B,), jnp.float32),
        scratch_types=[
            pltpu.VMEM((bw * 2 * D,), jnp.float32),
            pltpu.VMEM((bw,), jnp.float32),
            pltpu.VMEM((L * L,), jnp.float32),
        ],
        compiler_params=pltpu.CompilerParams(needs_layout_passes=False),
    )
    def k2(stage_hbm, out_hbm, flat_v, out_v, accbuf_v):
        wid = lax.axis_index("s") * num_cores + lax.axis_index("c")
        base = wid * bw
        pltpu.sync_copy(stage_hbm.at[pl.ds(base * 2 * D, bw * 2 * D)], flat_v)
        riota = lax.iota(jnp.int32, L)

        def body(g, carry):
            for j in range(L):
                p = (g * L + j) * 2 * D
                acc = flat_v[pl.ds(p, L)] * flat_v[pl.ds(p + D, L)]
                for q in range(1, D // L):
                    acc = acc + (flat_v[pl.ds(p + q * L, L)]
                                 * flat_v[pl.ds(p + D + q * L, L)])
                accbuf_v[pl.ds(j * L, L)] = acc
            res = jnp.zeros((L,), jnp.float32)
            for i in range(L):
                res = res + plsc.load_gather(accbuf_v, [riota * L + i])
            out_v[pl.ds(g * L, L)] = res
            return carry

        lax.fori_loop(0, bw // L, body, 0)
        pltpu.sync_copy(out_v, out_hbm.at[pl.ds(base, bw)])

    return k2


def kernel(inputs, user_table, movie_table):
    info = plsc.get_sparse_core_info()
    k1 = _make_scan_kernel(info.num_cores, info.num_subcores)
    k2 = _make_dot_kernel(info.num_cores, info.num_subcores)
    user_idx = inputs[:, 0]
    movie_idx = inputs[:, 1]
    stage = k1(user_idx, movie_idx, user_table.T, movie_table.T)
    out = k2(stage)
    return out.reshape(B, 1)


# speedup vs baseline: 3.5396x; 1.7307x over previous
import functools

import jax
import jax.numpy as jnp
from jax import lax
from jax.experimental import pallas as pl
from jax.experimental.pallas import tpu as pltpu
from jax.experimental.pallas import tpu_sc as plsc

B = 16384
D = 64
L = 16
CH = 256


def _make_sc_kernel(num_cores, num_subcores):
    NW = num_cores * num_subcores
    bw = B // NW
    mesh = plsc.VectorSubcoreMesh(core_axis_name="c", subcore_axis_name="s")

    @functools.partial(
        pl.kernel,
        mesh=mesh,
        out_type=jax.ShapeDtypeStruct((B,), jnp.float32),
        scratch_types=[
            pltpu.VMEM((bw,), jnp.int32),
            pltpu.VMEM((bw,), jnp.int32),
            pltpu.VMEM((CH, D), jnp.float32),
            pltpu.VMEM((CH, D), jnp.float32),
            pltpu.VMEM((bw,), jnp.float32),
            pltpu.VMEM((L * L,), jnp.float32),
            pltpu.SemaphoreType.DMA,
            pltpu.SemaphoreType.DMA,
        ],
        compiler_params=pltpu.CompilerParams(needs_layout_passes=False),
    )
    def k(uidx_hbm, midx_hbm, ut_hbm, mt_hbm, out_hbm,
          uidx_v, midx_v, urows_v, mrows_v, out_v, accbuf_v,
          sem_u, sem_m):
        wid = lax.axis_index("s") * num_cores + lax.axis_index("c")
        base = wid * bw
        pltpu.sync_copy(uidx_hbm.at[pl.ds(base, bw)], uidx_v)
        pltpu.sync_copy(midx_hbm.at[pl.ds(base, bw)], midx_v)

        riota = lax.iota(jnp.int32, L)

        def chunk(c, carry):
            c0 = c * CH

            def issue(g, carry2):
                ivu = uidx_v[pl.ds(c0 + g * L, L)]
                ivm = midx_v[pl.ds(c0 + g * L, L)]
                for j in range(L):
                    ru = ivu[j]
                    rm = ivm[j]
                    pltpu.make_async_copy(
                        ut_hbm.at[pl.ds(ru, 1)],
                        urows_v.at[pl.ds(g * L + j, 1)], sem_u).start()
                    pltpu.make_async_copy(
                        mt_hbm.at[pl.ds(rm, 1)],
                        mrows_v.at[pl.ds(g * L + j, 1)], sem_m).start()
                return carry2

            lax.fori_loop(0, CH // L, issue, 0)

            def drain(j, carry2):
                pltpu.make_async_copy(
                    ut_hbm.at[pl.ds(0, 1)], urows_v.at[pl.ds(0, 1)],
                    sem_u).wait()
                pltpu.make_async_copy(
                    mt_hbm.at[pl.ds(0, 1)], mrows_v.at[pl.ds(0, 1)],
                    sem_m).wait()
                return carry2

            lax.fori_loop(0, CH, drain, 0)

            def body(g, carry2):
                for j in range(L):
                    r = g * L + j
                    acc = urows_v[r, pl.ds(0, L)] * mrows_v[r, pl.ds(0, L)]
                    for kk in range(1, D // L):
                        acc = acc + (urows_v[r, pl.ds(kk * L, L)]
                                     * mrows_v[r, pl.ds(kk * L, L)])
                    accbuf_v[pl.ds(j * L, L)] = acc
                res = jnp.zeros((L,), jnp.float32)
                for i in range(L):
                    res = res + plsc.load_gather(accbuf_v, [riota * L + i])
                out_v[pl.ds(c0 + g * L, L)] = res
                return carry2

            lax.fori_loop(0, CH // L, body, 0)
            return carry

        lax.fori_loop(0, bw // CH, chunk, 0)
        pltpu.sync_copy(out_v, out_hbm.at[pl.ds(base, bw)])

    return k


def kernel(inputs, user_table, movie_table):
    info = plsc.get_sparse_core_info()
    k = _make_sc_kernel(info.num_cores, info.num_subcores)
    user_idx = inputs[:, 0]
    movie_idx = inputs[:, 1]
    out = k(user_idx, movie_idx, user_table, movie_table)
    return out.reshape(B, 1)
